# baseline (speedup 1.0000x reference)
import jax
import jax.numpy as jnp
from jax import lax
from jax.experimental import pallas as pl
from jax.experimental.pallas import tpu as pltpu
from jax.experimental.pallas import tpu_sc as plsc

D = 64
TROWS = 257
LQ = 2048
LK = 2048
E_ROWS = 4096
Q_PER_CORE = 1024
BAND_LO = 1919


def _scs_body(table_hbm, out_hbm, e_sh, sem):
    c = lax.axis_index("c")

    pltpu.sync_copy(table_hbm, e_sh.at[pl.ds(BAND_LO, TROWS)])
    pltpu.sync_copy(table_hbm.at[pl.ds(0, 1)], e_sh.at[pl.ds(0, 1)])
    n = 1
    while n < BAND_LO:
        m = min(n, BAND_LO - n)
        pltpu.sync_copy(e_sh.at[pl.ds(0, m)], e_sh.at[pl.ds(n, m)])
        n += m
    hi0 = BAND_LO + TROWS
    hi_len = E_ROWS - hi0
    pltpu.sync_copy(table_hbm.at[pl.ds(TROWS - 1, 1)], e_sh.at[pl.ds(hi0, 1)])
    n = 1
    while n < hi_len:
        m = min(n, hi_len - n)
        pltpu.sync_copy(e_sh.at[pl.ds(hi0, m)], e_sh.at[pl.ds(hi0 + n, m)])
        n += m

    q0 = c * Q_PER_CORE

    LEAD = 16

    def fire_row(i, _):
        pltpu.async_copy(e_sh.at[pl.ds((LQ - 1) - (q0 + i), LK)],
                         out_hbm.at[q0 + i], sem)

        @pl.when(i >= LEAD)
        def _():
            pltpu.make_async_copy(e_sh.at[pl.ds(0, LK)],
                                  out_hbm.at[q0], sem).wait()

        return 0

    lax.fori_loop(0, Q_PER_CORE, fire_row, 0)

    def drain_row(i, _):
        pltpu.make_async_copy(e_sh.at[pl.ds(0, LK)], out_hbm.at[q0], sem).wait()
        return 0

    lax.fori_loop(0, LEAD, drain_row, 0)


def kernel(length_q, length_k, embeddings_table):
    mesh = plsc.ScalarSubcoreMesh(axis_name="c")
    call = pl.kernel(
        _scs_body,
        out_type=jax.ShapeDtypeStruct((LQ, LK, D), jnp.float32),
        mesh=mesh,
        scratch_types=[
            pltpu.VMEM_SHARED((E_ROWS, D), jnp.float32),
            pltpu.SemaphoreType.DMA,
        ],
        compiler_params=pltpu.CompilerParams(use_tc_tiling_on_sc=False),
    )
    return call(embeddings_table)
